# Initial kernel scaffold; baseline (speedup 1.0000x reference)
import jax
import jax.numpy as jnp
from jax.experimental import pallas as pl


def kernel(proj_online, labels_online, proj_queue, labels_queue):
    raise NotImplementedError("write your pallas kernel here")


# speedup vs baseline: 2.3793x; 2.3793x over previous
import functools

import jax
import jax.numpy as jnp
from jax.experimental import pallas as pl
from jax.experimental.pallas import tpu as pltpu

_TOP_M = 20
_INV_Q = 10
_MARGIN = 0.1
_BETA = 20.0
_SLOTS = 32


def _body(nb, po_ref, pq_ref, loc_ref, lor_ref, lq_ref, out_ref,
          x_ref, rv_ref, rc_ref, nc_ref):
    i = pl.program_id(0)
    B = po_ref.shape[0]

    po = po_ref[...]
    pq = pq_ref[...]
    s = jax.lax.dot_general(po, pq, (((1,), (1,)), ((), ())),
                            preferred_element_type=jnp.float32)
    lo = loc_ref[...]
    lq = lq_ref[...]
    same = lo == lq
    x_ref[...] = jnp.where(same, -jnp.inf, s)
    negc = jnp.sum(jnp.where(same, 0.0, 1.0), axis=1, keepdims=True)

    nc_ref[...] = jnp.where(i == 0, jnp.zeros_like(negc), nc_ref[...]) + negc

    bvals, bcnts = [], []
    for k in range(_TOP_M):
        x = x_ref[...]
        m = jnp.max(x, axis=1, keepdims=True)
        eq = x == m
        c = jnp.sum(jnp.where(eq, 1.0, 0.0), axis=1, keepdims=True)
        bvals.append(m)
        bcnts.append(c)
        if k + 1 < _TOP_M:
            x_ref[...] = jnp.where(eq, -jnp.inf, x)

    ninf = jnp.full((B, _SLOTS), -jnp.inf, jnp.float32)
    prev_v = jnp.where(i == 0, ninf, rv_ref[...])
    prev_c = jnp.where(i == 0, jnp.zeros_like(ninf), rc_ref[...])
    allv = jnp.concatenate([prev_v] + bvals, axis=1)
    allc = jnp.concatenate([prev_c] + bcnts, axis=1)
    mvals, mcnts = [], []
    for k in range(_TOP_M):
        m = jnp.max(allv, axis=1, keepdims=True)
        eqm = allv == m
        c = jnp.sum(jnp.where(eqm, allc, 0.0), axis=1, keepdims=True)
        mvals.append(m)
        mcnts.append(c)
        if k + 1 < _TOP_M:
            allv = jnp.where(eqm, -jnp.inf, allv)
    pad_v = jnp.full((B, _SLOTS - _TOP_M), -jnp.inf, jnp.float32)
    pad_c = jnp.zeros((B, _SLOTS - _TOP_M), jnp.float32)
    rv_ref[...] = jnp.concatenate(mvals + [pad_v], axis=1)
    rc_ref[...] = jnp.concatenate(mcnts + [pad_c], axis=1)

    @pl.when(i == nb - 1)
    def _final():
        nvn = nc_ref[...]

        mx = mvals[0]
        mx0 = jnp.where(mx == -jnp.inf, 0.0, mx)
        rem = jnp.full((B, 1), float(_TOP_M), jnp.float32)
        acc = jnp.zeros((B, 1), jnp.float32)
        for k in range(_TOP_M):
            w = jnp.minimum(mcnts[k], rem)
            rem = rem - w
            d = jnp.where(mvals[k] == -jnp.inf, -jnp.inf, mvals[k] - mx0)
            acc = acc + w * jnp.exp(_BETA * d)
        lse = _BETA * mx0 + jnp.log(jnp.maximum(acc, 1e-30))
        n_tail = lse / _BETA
        m_per = jnp.clip(nvn, 0.0, float(_TOP_M))
        n_tail = n_tail - jnp.log(jnp.maximum(m_per, 1.0)) / _BETA

        simb = jax.lax.dot_general(po, po, (((1,), (1,)), ((), ())),
                                   preferred_element_type=jnp.float32)
        lor = lor_ref[...]
        r_i = jax.lax.broadcasted_iota(jnp.int32, (B, B), 0)
        c_j = jax.lax.broadcasted_iota(jnp.int32, (B, B), 1)
        diag = r_i == c_j
        posm = (lo == lor) & (~diag)
        has_pos = jnp.max(jnp.where(posm, 1.0, 0.0), axis=1, keepdims=True)
        hardest = jnp.min(jnp.where(posm, simb, jnp.inf), axis=1,
                          keepdims=True)

        has_neg = jnp.where(nvn >= 1.0, 1.0, 0.0)
        valid = has_pos * has_neg
        z = n_tail - hardest + _MARGIN
        sp = jnp.maximum(z, 0.0) + jnp.log(1.0 + jnp.exp(-jnp.abs(z)))
        loss = sp * valid

        n_valid = jnp.sum(valid, axis=(0, 1), keepdims=True)
        n_sel = jnp.maximum(1.0, jnp.floor(n_valid / float(_INV_Q)))

        eye = jnp.where(diag, 1.0, 0.0)
        lrow = jax.lax.dot_general(loss, eye, (((0,), (0,)), ((), ())))
        gt = jnp.where(lrow > loss, 1.0, 0.0)
        tie = jnp.where((lrow == loss) & (c_j < r_i), 1.0, 0.0)
        rank = jnp.sum(gt + tie, axis=1, keepdims=True)
        sel = jnp.where(rank < n_sel, 1.0, 0.0)
        total = jnp.sum(loss * sel, axis=(0, 1), keepdims=True)
        cvar = total / n_sel
        out_ref[...] = jnp.where(n_valid >= 2.0, cvar, jnp.zeros((1, 1)))


def kernel(proj_online, labels_online, proj_queue, labels_queue):
    B, D = proj_online.shape
    Q = proj_queue.shape[0]
    W = 2048 if (Q % 2048 == 0) else Q
    nb = Q // W

    lo_col = labels_online.reshape(B, 1)
    lo_row = labels_online.reshape(1, B)
    lq_row = labels_queue.reshape(1, Q)

    body = functools.partial(_body, nb)

    out = pl.pallas_call(
        body,
        grid=(nb,),
        in_specs=[
            pl.BlockSpec((B, D), lambda i: (0, 0)),
            pl.BlockSpec((W, D), lambda i: (i, 0)),
            pl.BlockSpec((B, 1), lambda i: (0, 0)),
            pl.BlockSpec((1, B), lambda i: (0, 0)),
            pl.BlockSpec((1, W), lambda i: (0, i)),
        ],
        out_specs=pl.BlockSpec((1, 1), lambda i: (0, 0)),
        out_shape=jax.ShapeDtypeStruct((1, 1), jnp.float32),
        scratch_shapes=[
            pltpu.VMEM((B, W), jnp.float32),
            pltpu.VMEM((B, _SLOTS), jnp.float32),
            pltpu.VMEM((B, _SLOTS), jnp.float32),
            pltpu.VMEM((B, 1), jnp.float32),
        ],
        compiler_params=pltpu.CompilerParams(
            dimension_semantics=("arbitrary",)),
    )(proj_online, proj_queue, lo_col, lo_row, lq_row)
    return out[0, 0]


# speedup vs baseline: 7.1318x; 2.9974x over previous
import functools

import jax
import jax.numpy as jnp
from jax import lax
from jax.experimental import pallas as pl
from jax.experimental.pallas import tpu as pltpu
from jax.experimental.pallas import tpu_sc as plsc

_TOP_M = 20
_INV_Q = 10
_MARGIN = 0.1
_BETA = 20.0
_CHW = 128
_GIDX_SLOTS = 32


def _k1_body(nb, nch_blk, po_ref, pq_ref, loc_ref, lq_ref,
             sim_ref, cmax_ref, neg_ref, nc_ref):
    i = pl.program_id(0)
    po = po_ref[...]
    pq = pq_ref[...]
    s = lax.dot_general(po, pq, (((1,), (1,)), ((), ())),
                        preferred_element_type=jnp.float32)
    same = loc_ref[...] == lq_ref[...]
    x = jnp.where(same, -jnp.inf, s)
    sim_ref[...] = x
    negc = jnp.sum(jnp.where(same, 0.0, 1.0), axis=1, keepdims=True)
    nc_ref[...] = jnp.where(i == 0, jnp.zeros_like(negc), nc_ref[...]) + negc
    cms = [jnp.max(x[:, c * _CHW:(c + 1) * _CHW], axis=1, keepdims=True)
           for c in range(nch_blk)]
    cmax_ref[0] = jnp.concatenate(cms, axis=1)

    @pl.when(i == nb - 1)
    def _():
        neg_ref[...] = nc_ref[...]


def _k1(po, pq, lo_col, lq_row, W):
    B, D = po.shape
    Q = pq.shape[0]
    nb = Q // W
    nch_blk = W // _CHW
    return pl.pallas_call(
        functools.partial(_k1_body, nb, nch_blk),
        grid=(nb,),
        in_specs=[
            pl.BlockSpec((B, D), lambda i: (0, 0)),
            pl.BlockSpec((W, D), lambda i: (i, 0)),
            pl.BlockSpec((B, 1), lambda i: (0, 0)),
            pl.BlockSpec((1, W), lambda i: (0, i)),
        ],
        out_specs=[
            pl.BlockSpec((B, W), lambda i: (0, i)),
            pl.BlockSpec((1, B, nch_blk), lambda i: (i, 0, 0)),
            pl.BlockSpec((B, 1), lambda i: (0, 0)),
        ],
        out_shape=[
            jax.ShapeDtypeStruct((B, Q), jnp.float32),
            jax.ShapeDtypeStruct((nb, B, nch_blk), jnp.float32),
            jax.ShapeDtypeStruct((B, 1), jnp.float32),
        ],
        scratch_shapes=[pltpu.VMEM((B, 1), jnp.float32)],
        compiler_params=pltpu.CompilerParams(
            dimension_semantics=("arbitrary",)),
    )(po, pq, lo_col, lq_row)


def _k2_body(nsel, nch, po_ref, cm_ref, loc_ref, lor_ref,
             gidx_ref, hard_ref, hpos_ref):
    B = po_ref.shape[0]
    x = cm_ref[...]
    col = lax.broadcasted_iota(jnp.int32, (B, nch), 1)
    ids = []
    for k in range(nsel):
        m = jnp.max(x, axis=1, keepdims=True)
        eq = x == m
        idx = jnp.min(jnp.where(eq, col, jnp.int32(2 ** 30)), axis=1,
                      keepdims=True)
        ids.append(idx)
        if k + 1 < nsel:
            x = jnp.where(col == idx, -jnp.inf, x)
    ids = ids + [ids[0]] * (_GIDX_SLOTS - nsel)
    row = lax.broadcasted_iota(jnp.int32, (B, 1), 0)
    gidx_ref[...] = jnp.concatenate(ids, axis=1) + row * nch

    po = po_ref[...]
    simb = lax.dot_general(po, po, (((1,), (1,)), ((), ())),
                           preferred_element_type=jnp.float32)
    r_i = lax.broadcasted_iota(jnp.int32, (B, B), 0)
    c_j = lax.broadcasted_iota(jnp.int32, (B, B), 1)
    posm = (loc_ref[...] == lor_ref[...]) & (r_i != c_j)
    hpos_ref[...] = jnp.max(jnp.where(posm, 1.0, 0.0), axis=1, keepdims=True)
    hard_ref[...] = jnp.min(jnp.where(posm, simb, jnp.inf), axis=1,
                            keepdims=True)


def _k2(po, cmaxT, lo_col, lo_row, nsel):
    B, D = po.shape
    nch = cmaxT.shape[1]
    return pl.pallas_call(
        functools.partial(_k2_body, nsel, nch),
        out_shape=[
            jax.ShapeDtypeStruct((B, _GIDX_SLOTS), jnp.int32),
            jax.ShapeDtypeStruct((B, 1), jnp.float32),
            jax.ShapeDtypeStruct((B, 1), jnp.float32),
        ],
    )(po, cmaxT, lo_col, lo_row)


def _k3(simv, gidx, B):
    RPW = B // 32
    G = 4
    NGRP = RPW // G
    NR = G * _GIDX_SLOTS

    mesh = plsc.VectorSubcoreMesh(core_axis_name="c", subcore_axis_name="s")
    gidx_flat = gidx.reshape(-1)

    @functools.partial(
        pl.kernel,
        out_type=jax.ShapeDtypeStruct((B * _GIDX_SLOTS, _CHW), jnp.float32),
        mesh=mesh,
        scratch_types=[
            pltpu.VMEM((RPW * _GIDX_SLOTS,), jnp.int32),
            pltpu.VMEM((2, NR, _CHW), jnp.float32),
            pltpu.SemaphoreType.DMA,
        ],
    )
    def k3(simv_hbm, gidx_hbm, out_hbm, idx_v, chunks_v, gsem):
        ncores = 2
        w = lax.axis_index("s") * ncores + lax.axis_index("c")
        base = w * RPW

        pltpu.sync_copy(gidx_hbm.at[pl.ds(base * _GIDX_SLOTS,
                                          RPW * _GIDX_SLOTS)], idx_v)

        def start(g, slot):
            pltpu.async_copy(
                simv_hbm.at[idx_v.at[pl.ds(g * NR, NR)]],
                chunks_v.at[slot], gsem)

        start(0, 0)

        def group_loop(g, carry):
            slot = lax.rem(g, 2)
            pltpu.make_async_copy(
                simv_hbm.at[idx_v.at[pl.ds(0, NR)]],
                chunks_v.at[slot], gsem).wait()

            @pl.when(g + 1 < NGRP)
            def _():
                start(g + 1, lax.rem(g + 1, 2))

            pltpu.sync_copy(
                chunks_v.at[slot],
                out_hbm.at[pl.ds((base + g * G) * _GIDX_SLOTS, NR)])
            return carry
        lax.fori_loop(0, NGRP, group_loop, 0)

    return k3(simv, gidx_flat)


def _k4_body(nsel, simg_ref, neg_ref, hard_ref, hpos_ref, out_ref):
    B = simg_ref.shape[0]
    S = simg_ref.shape[1]
    nvn = neg_ref[...]

    col = lax.broadcasted_iota(jnp.int32, (B, S), 1)
    x = jnp.where(col < nsel * _CHW, simg_ref[...], -jnp.inf)
    mx = jnp.max(x, axis=1, keepdims=True)
    mx0 = jnp.where(mx == -jnp.inf, 0.0, mx)
    rem = jnp.full((B, 1), float(_TOP_M), jnp.float32)
    acc = jnp.zeros((B, 1), jnp.float32)
    for k in range(_TOP_M):
        m = jnp.max(x, axis=1, keepdims=True)
        eq = x == m
        c = jnp.sum(jnp.where(eq, 1.0, 0.0), axis=1, keepdims=True)
        w = jnp.minimum(c, rem)
        rem = rem - w
        d = jnp.where(m == -jnp.inf, -jnp.inf, m - mx0)
        acc = acc + w * jnp.exp(_BETA * d)
        if k + 1 < _TOP_M:
            x = jnp.where(eq, -jnp.inf, x)
    lse = _BETA * mx0 + jnp.log(jnp.maximum(acc, 1e-30))
    n_tail = lse / _BETA
    m_per = jnp.clip(nvn, 0.0, float(_TOP_M))
    n_tail = n_tail - jnp.log(jnp.maximum(m_per, 1.0)) / _BETA

    hardest = hard_ref[...]
    valid = hpos_ref[...] * jnp.where(nvn >= 1.0, 1.0, 0.0)
    z = n_tail - hardest + _MARGIN
    sp = jnp.maximum(z, 0.0) + jnp.log(1.0 + jnp.exp(-jnp.abs(z)))
    loss = sp * valid

    n_valid = jnp.sum(valid, axis=(0, 1), keepdims=True)
    n_sel = jnp.maximum(1.0, jnp.floor(n_valid / float(_INV_Q)))

    r_i = lax.broadcasted_iota(jnp.int32, (B, B), 0)
    c_j = lax.broadcasted_iota(jnp.int32, (B, B), 1)
    eye = jnp.where(r_i == c_j, 1.0, 0.0)
    lrow = lax.dot_general(loss, eye, (((0,), (0,)), ((), ())))
    gt = jnp.where(lrow > loss, 1.0, 0.0)
    tie = jnp.where((lrow == loss) & (c_j < r_i), 1.0, 0.0)
    rank = jnp.sum(gt + tie, axis=1, keepdims=True)
    sel = jnp.where(rank < n_sel, 1.0, 0.0)
    total = jnp.sum(loss * sel, axis=(0, 1), keepdims=True)
    out_ref[...] = jnp.where(n_valid >= 2.0, total / n_sel,
                             jnp.zeros((1, 1)))


def _k4(simg, neg, hard, hpos, nsel):
    return pl.pallas_call(
        functools.partial(_k4_body, nsel),
        out_shape=jax.ShapeDtypeStruct((1, 1), jnp.float32),
    )(simg, neg, hard, hpos)


def kernel(proj_online, labels_online, proj_queue, labels_queue):
    B, D = proj_online.shape
    Q = proj_queue.shape[0]
    W = 2048 if Q % 2048 == 0 else Q
    nch = Q // _CHW
    nsel = min(_TOP_M, nch)

    lo_col = labels_online.reshape(B, 1)
    lo_row = labels_online.reshape(1, B)
    lq_row = labels_queue.reshape(1, Q)

    sim, cmax, neg = _k1(proj_online, proj_queue, lo_col, lq_row, W)
    cmaxT = jnp.transpose(cmax, (1, 0, 2)).reshape(B, nch)
    gidx, hard, hpos = _k2(proj_online, cmaxT, lo_col, lo_row, nsel)
    simv = sim.reshape(B * nch, _CHW)
    simg = _k3(simv, gidx, B)
    out = _k4(simg.reshape(B, _GIDX_SLOTS * _CHW), neg, hard, hpos, nsel)
    return out[0, 0]
